# baseline (speedup 1.0000x reference)
import functools

import jax
import jax.numpy as jnp
from jax import lax
from jax.experimental import pallas as pl
from jax.experimental.pallas import tpu as pltpu
from jax.experimental.pallas import tpu_sc as plsc

NUM_POINTS = 1000000
DIM = 64
BATCH = 16384

P_STREAM = 999936
CHUNK = 128
N_CHUNKS = P_STREAM // CHUNK
HALF = 122
ROWCAP = 352
STAGE = 712
OUT_ROWS = BATCH + 32


def kernel(idx, weight):
    info = plsc.get_sparse_core_info()

    mesh = plsc.VectorSubcoreMesh(core_axis_name="c", subcore_axis_name="s")

    @functools.partial(
        pl.kernel,
        mesh=mesh,
        out_type=jax.ShapeDtypeStruct((OUT_ROWS, 2 * DIM), jnp.float32),
        scratch_types=[
            pltpu.VMEM((2048,), jnp.int32),
            pltpu.VMEM((STAGE,), jnp.int32),
            pltpu.VMEM((STAGE,), jnp.int32),
            pltpu.VMEM((ROWCAP,), jnp.int32),
        ] + [pltpu.VMEM((DIM, CHUNK), jnp.float32)] * 8
          + [pltpu.VMEM((ROWCAP, 2 * DIM), jnp.float32)]
          + [pltpu.SemaphoreType.DMA] * 8,
        compiler_params=pltpu.CompilerParams(
            use_tc_tiling_on_sc=True, needs_layout_passes=False
        ),
    )
    def gather_kernel(sp_hbm, so_hbm, wt_hbm, tail_hbm, out_hbm, scan_b,
                      seg_p, seg_j, slot_v, buf0, buf1, buf2, buf3, buf4,
                      buf5, buf6, buf7, rows_b, sem0, sem1, sem2, sem3,
                      sem4, sem5, sem6, sem7):
        wid = lax.axis_index("s") * info.num_cores + lax.axis_index("c")
        c0 = 244 * wid + jnp.minimum(wid, 4)
        nck = 244 + jnp.where(wid < 4, 1, 0)
        my_start = c0 * CHUNK
        zeros16 = jnp.zeros((16,), jnp.int32)

        def piece(p, s):
            pltpu.sync_copy(sp_hbm.at[pl.ds(p * 2048, 2048)], scan_b)

            def vv(k, s2):
                v = scan_b[pl.ds(k * 16, 16)]
                return s2 + jnp.sum((v < my_start).astype(jnp.int32))

            return lax.fori_loop(0, 128, vv, s)

        lo = lax.fori_loop(0, 8, piece, jnp.int32(0))
        lo8 = pl.multiple_of((lo // 8) * 8, 8)
        pltpu.sync_copy(sp_hbm.at[pl.ds(lo8, STAGE)], seg_p)
        pltpu.sync_copy(so_hbm.at[pl.ds(lo8, STAGE)], seg_j)

        trash = BATCH + wid

        def prefill(k, c):
            slot_v[pl.ds(k * 16, 16)] = zeros16 + trash
            return c

        def process(pbase, size, buf, g, mc):
            def step(k, st):
                gg, mcc, act = st
                off = pl.multiple_of(gg * 16, 16)
                pv = seg_p[pl.ds(off, 16)]
                jv = seg_j[pl.ds(off, 16)]
                below = pv < (pbase + size)
                m = (pv >= pbase) & below & act
                mi = m.astype(jnp.int32)
                cnt = jnp.sum(mi)
                pref = plsc.cumsum(mi) - 1
                dst = jnp.minimum(mcc + pref, ROWCAP - 1)
                col = pv - pbase

                @pl.when(cnt > 0)
                def _():
                    plsc.store_scatter(slot_v, [dst], jv, mask=m)

                    def dl(d, c2):
                        vals = plsc.load_gather(buf, [zeros16 + d, col],
                                                mask=m)
                        plsc.store_scatter(rows_b, [dst, zeros16 + d], vals,
                                           mask=m)
                        return c2

                    lax.fori_loop(0, DIM, dl, 0)

                adv = act & (jnp.sum(below.astype(jnp.int32)) == 16)
                g2 = jnp.where(adv, jnp.minimum(gg + 1, STAGE // 16 - 2), gg)
                return (g2, mcc + cnt, adv)

            g, mc, _ = lax.fori_loop(0, 8, step, (g, mc, jnp.bool_(True)))
            return g, mc

        def run_half(h, g):
            cbeg = HALF * h
            cend = jnp.minimum(jnp.int32(HALF), nck) if h == 0 else nck
            bufs = ((buf0, sem0), (buf1, sem1), (buf2, sem2), (buf3, sem3),
                    (buf4, sem4), (buf5, sem5), (buf6, sem6), (buf7, sem7))
            lax.fori_loop(0, ROWCAP // 16, prefill, 0)
            for b in range(8):
                pltpu.make_async_copy(
                    wt_hbm.at[:, pl.ds(
                        pl.multiple_of((c0 + cbeg + b) * CHUNK, CHUNK),
                        CHUNK)],
                    bufs[b][0], bufs[b][1]).start()

            def quad(t, st):
                g, mc = st
                for b, (buf, sem) in enumerate(bufs):
                    c = cbeg + 8 * t + b

                    @pl.when(c < cend)
                    def _():
                        pltpu.make_async_copy(
                            wt_hbm.at[:, pl.ds(0, CHUNK)], buf, sem).wait()

                    valid = c < cend
                    pbase = jnp.where(valid, (c0 + c) * CHUNK, 0)
                    size = jnp.where(valid, CHUNK, 0)
                    g, mc = process(pbase, size, buf, g, mc)

                    @pl.when(c + 8 < cend)
                    def _():
                        pltpu.make_async_copy(
                            wt_hbm.at[:, pl.ds(
                                pl.multiple_of((c0 + c + 8) * CHUNK, CHUNK),
                                CHUNK)],
                            buf, sem).start()
                return (g, mc)

            g, mc = lax.fori_loop(0, (HALF + 7) // 8, quad,
                                  (g, jnp.int32(0)))
            return g, mc

        g = jnp.int32(0)
        g, _ = run_half(0, g)
        pltpu.async_copy(rows_b, out_hbm.at[slot_v], sem0).wait()

        g, mc = run_half(1, g)
        pltpu.sync_copy(tail_hbm, buf0.at[:, pl.ds(0, 2 * DIM)])
        g, mc = process(jnp.int32(P_STREAM), jnp.int32(2 * DIM), buf0, g, mc)
        pltpu.async_copy(rows_b, out_hbm.at[slot_v], sem0).wait()

    idx32 = idx.astype(jnp.int32)
    sp, order = lax.sort_key_val(idx32, jnp.arange(BATCH, dtype=jnp.int32))
    pad_i = jnp.full((STAGE + 24,), 1 << 30, jnp.int32)
    sp_pad = jnp.concatenate([sp, pad_i])
    so_pad = jnp.concatenate([order, jnp.zeros((STAGE + 24,), jnp.int32)])
    wt_tail = jnp.pad(weight[P_STREAM:].T,
                      ((0, 0), (0, 2 * DIM - (NUM_POINTS - P_STREAM))))
    out128 = gather_kernel(sp_pad, so_pad, weight.T, wt_tail)
    return out128[:BATCH, :DIM]


# speedup vs baseline: 1.3281x; 1.3281x over previous
import functools

import jax
import jax.numpy as jnp
from jax import lax
from jax.experimental import pallas as pl
from jax.experimental.pallas import tpu as pltpu
from jax.experimental.pallas import tpu_sc as plsc

NUM_POINTS = 1000000
DIM = 64
BATCH = 16384

P_STREAM = 999936
CHUNK = 256
N_CHUNKS = P_STREAM // CHUNK
HALF = 61
ROWCAP = 352
STAGE = 712
OUT_ROWS = BATCH + 32


def kernel(idx, weight):
    info = plsc.get_sparse_core_info()

    mesh = plsc.VectorSubcoreMesh(core_axis_name="c", subcore_axis_name="s")

    @functools.partial(
        pl.kernel,
        mesh=mesh,
        out_type=jax.ShapeDtypeStruct((OUT_ROWS, 2 * DIM), jnp.float32),
        scratch_types=[
            pltpu.VMEM((2048,), jnp.int32),
            pltpu.VMEM((STAGE,), jnp.int32),
            pltpu.VMEM((STAGE,), jnp.int32),
            pltpu.VMEM((ROWCAP,), jnp.int32),
            pltpu.VMEM((DIM, CHUNK), jnp.float32),
            pltpu.VMEM((DIM, CHUNK), jnp.float32),
            pltpu.VMEM((DIM, CHUNK), jnp.float32),
            pltpu.VMEM((DIM, CHUNK), jnp.float32),
            pltpu.VMEM((ROWCAP, 2 * DIM), jnp.float32),
            pltpu.SemaphoreType.DMA,
            pltpu.SemaphoreType.DMA,
            pltpu.SemaphoreType.DMA,
            pltpu.SemaphoreType.DMA,
        ],
        compiler_params=pltpu.CompilerParams(
            use_tc_tiling_on_sc=True, needs_layout_passes=False
        ),
    )
    def gather_kernel(sp_hbm, so_hbm, wt_hbm, tail_hbm, out_hbm, scan_b,
                      seg_p, seg_j, slot_v, buf0, buf1, buf2, buf3, rows_b,
                      sem0, sem1, sem2, sem3):
        wid = lax.axis_index("s") * info.num_cores + lax.axis_index("c")
        c0 = 122 * wid + jnp.minimum(wid, 2)
        nck = 122 + jnp.where(wid < 2, 1, 0)
        my_start = c0 * CHUNK
        zeros16 = jnp.zeros((16,), jnp.int32)

        def piece(p, s):
            pltpu.sync_copy(sp_hbm.at[pl.ds(p * 2048, 2048)], scan_b)

            def vv(k, s2):
                v = scan_b[pl.ds(k * 16, 16)]
                return s2 + jnp.sum((v < my_start).astype(jnp.int32))

            return lax.fori_loop(0, 128, vv, s)

        lo = lax.fori_loop(0, 8, piece, jnp.int32(0))
        lo8 = pl.multiple_of((lo // 8) * 8, 8)
        pltpu.sync_copy(sp_hbm.at[pl.ds(lo8, STAGE)], seg_p)
        pltpu.sync_copy(so_hbm.at[pl.ds(lo8, STAGE)], seg_j)

        trash = BATCH + wid

        def prefill(k, c):
            slot_v[pl.ds(k * 16, 16)] = zeros16 + trash
            return c

        def process(pbase, size, buf, g, mc):
            def step(k, st):
                gg, mcc, act = st
                off = pl.multiple_of(gg * 16, 16)
                pv = seg_p[pl.ds(off, 16)]
                jv = seg_j[pl.ds(off, 16)]
                below = pv < (pbase + size)
                m = (pv >= pbase) & below & act
                mi = m.astype(jnp.int32)
                cnt = jnp.sum(mi)
                pref = plsc.cumsum(mi) - 1
                dst = jnp.minimum(mcc + pref, ROWCAP - 1)
                col = pv - pbase

                @pl.when(cnt > 0)
                def _():
                    plsc.store_scatter(slot_v, [dst], jv, mask=m)

                    def dl(d, c2):
                        vals = plsc.load_gather(buf, [zeros16 + d, col],
                                                mask=m)
                        plsc.store_scatter(rows_b, [dst, zeros16 + d], vals,
                                           mask=m)
                        return c2

                    lax.fori_loop(0, DIM, dl, 0)

                adv = act & (jnp.sum(below.astype(jnp.int32)) == 16)
                g2 = jnp.where(adv, jnp.minimum(gg + 1, STAGE // 16 - 2), gg)
                return (g2, mcc + cnt, adv)

            g, mc, _ = lax.fori_loop(0, 8, step, (g, mc, jnp.bool_(True)))
            return g, mc

        def run_half(h, g):
            cbeg = HALF * h
            cend = jnp.minimum(jnp.int32(HALF), nck) if h == 0 else nck
            bufs = ((buf0, sem0), (buf1, sem1), (buf2, sem2), (buf3, sem3))
            lax.fori_loop(0, ROWCAP // 16, prefill, 0)
            for b in range(4):
                pltpu.make_async_copy(
                    wt_hbm.at[:, pl.ds(
                        pl.multiple_of((c0 + cbeg + b) * CHUNK, CHUNK),
                        CHUNK)],
                    bufs[b][0], bufs[b][1]).start()

            def quad(t, st):
                g, mc = st
                for b, (buf, sem) in enumerate(bufs):
                    c = cbeg + 4 * t + b

                    @pl.when(c < cend)
                    def _():
                        pltpu.make_async_copy(
                            wt_hbm.at[:, pl.ds(0, CHUNK)], buf, sem).wait()

                    valid = c < cend
                    pbase = jnp.where(valid, (c0 + c) * CHUNK, 0)
                    size = jnp.where(valid, CHUNK, 0)
                    g, mc = process(pbase, size, buf, g, mc)

                    @pl.when(c + 4 < cend)
                    def _():
                        pltpu.make_async_copy(
                            wt_hbm.at[:, pl.ds(
                                pl.multiple_of((c0 + c + 4) * CHUNK, CHUNK),
                                CHUNK)],
                            buf, sem).start()
                return (g, mc)

            g, mc = lax.fori_loop(0, (HALF + 3) // 4, quad,
                                  (g, jnp.int32(0)))
            return g, mc

        g = jnp.int32(0)
        g, _ = run_half(0, g)
        pltpu.async_copy(rows_b, out_hbm.at[slot_v], sem0).wait()

        g, mc = run_half(1, g)
        pltpu.sync_copy(tail_hbm, buf0.at[:, pl.ds(0, 2 * DIM)])
        g, mc = process(jnp.int32(P_STREAM), jnp.int32(2 * DIM), buf0, g, mc)
        pltpu.async_copy(rows_b, out_hbm.at[slot_v], sem0).wait()

    idx32 = idx.astype(jnp.int32)
    sp, order = lax.sort_key_val(idx32, jnp.arange(BATCH, dtype=jnp.int32))
    pad_i = jnp.full((STAGE + 24,), 1 << 30, jnp.int32)
    sp_pad = jnp.concatenate([sp, pad_i])
    so_pad = jnp.concatenate([order, jnp.zeros((STAGE + 24,), jnp.int32)])
    wt_tail = jnp.pad(weight[P_STREAM:].T,
                      ((0, 0), (0, 2 * DIM - (NUM_POINTS - P_STREAM))))
    out128 = gather_kernel(sp_pad, so_pad, weight.T, wt_tail)
    return out128[:BATCH, :DIM]
